# initial kernel scaffold (unmeasured)
import jax
import jax.numpy as jnp
from jax import lax
from jax.experimental import pallas as pl
from jax.experimental.pallas import tpu as pltpu


def kernel(
    x,
):
    def body(*refs):
        pass

    out_shape = jax.ShapeDtypeStruct(..., jnp.float32)
    return pl.pallas_call(body, out_shape=out_shape)(...)


# baseline (device time: 198926 ns/iter reference)
import jax
import jax.numpy as jnp
from jax import lax
from jax.experimental import pallas as pl
from jax.experimental.pallas import tpu as pltpu


def kernel(x):
    M, N = x.shape
    HALF = M // 2

    def body(x_hbm, out_ref, xf32, ysend, yrecv, load_sem, send_sems, recv_sems):
        my_x = lax.axis_index("x")
        my_y = lax.axis_index("y")
        y_nbr = (my_x, 1 - my_y)
        x_nbr = (1 - my_x, my_y)

        barrier_sem = pltpu.get_barrier_semaphore()
        for nbr in (y_nbr, x_nbr):
            pl.semaphore_signal(
                barrier_sem, inc=1,
                device_id=nbr, device_id_type=pl.DeviceIdType.MESH,
            )
        pl.semaphore_wait(barrier_sem, 2)

        row0 = my_x * HALF
        other_row0 = (1 - my_x) * HALF

        load = pltpu.make_async_copy(
            x_hbm.at[pl.ds(row0, HALF)], xf32, load_sem
        )
        load.start()
        load.wait()
        ysend[...] = xf32[...].astype(jnp.bfloat16)

        rdma_y = pltpu.make_async_remote_copy(
            src_ref=ysend,
            dst_ref=yrecv,
            send_sem=send_sems.at[0],
            recv_sem=recv_sems.at[0],
            device_id=y_nbr,
            device_id_type=pl.DeviceIdType.MESH,
        )
        rdma_y.start()
        rdma_y.wait()

        out_ref[pl.ds(row0, HALF), :] = ysend[...] + yrecv[...]

        rdma_x_send = pltpu.make_async_remote_copy(
            src_ref=out_ref.at[pl.ds(row0, HALF)],
            dst_ref=out_ref.at[pl.ds(row0, HALF)],
            send_sem=send_sems.at[1],
            recv_sem=recv_sems.at[1],
            device_id=x_nbr,
            device_id_type=pl.DeviceIdType.MESH,
        )
        rdma_x_send.start()
        rdma_x_send.wait_send()
        rdma_x_recv = pltpu.make_async_remote_copy(
            src_ref=out_ref.at[pl.ds(row0, HALF)],
            dst_ref=out_ref.at[pl.ds(other_row0, HALF)],
            send_sem=send_sems.at[1],
            recv_sem=recv_sems.at[1],
            device_id=x_nbr,
            device_id_type=pl.DeviceIdType.MESH,
        )
        rdma_x_recv.wait_recv()

    return pl.pallas_call(
        body,
        out_shape=jax.ShapeDtypeStruct((M, N), jnp.bfloat16),
        in_specs=[pl.BlockSpec(memory_space=pl.ANY)],
        out_specs=pl.BlockSpec(memory_space=pltpu.VMEM),
        scratch_shapes=[
            pltpu.VMEM((HALF, N), jnp.float32),
            pltpu.VMEM((HALF, N), jnp.bfloat16),
            pltpu.VMEM((HALF, N), jnp.bfloat16),
            pltpu.SemaphoreType.DMA,
            pltpu.SemaphoreType.DMA((2,)),
            pltpu.SemaphoreType.DMA((2,)),
        ],
        compiler_params=pltpu.CompilerParams(collective_id=0),
    )(x)


# device time: 113423 ns/iter; 1.7538x vs baseline; 1.7538x over previous
import jax
import jax.numpy as jnp
from jax import lax
from jax.experimental import pallas as pl
from jax.experimental.pallas import tpu as pltpu


def kernel(x):
    M, N = x.shape
    HALF = M // 2

    C = 16
    ROWS = HALF // C

    def body(x_hbm, out_ref, xf32, ysend, yrecv,
             load_sem, ysend_sems, yrecv_sems, xsend_sems, xrecv_sems):
        my_x = lax.axis_index("x")
        my_y = lax.axis_index("y")
        y_nbr = (my_x, 1 - my_y)
        x_nbr = (1 - my_x, my_y)

        row0 = my_x * HALF
        other_row0 = (1 - my_x) * HALF

        load = pltpu.make_async_copy(
            x_hbm.at[pl.ds(row0, HALF)], xf32, load_sem
        )
        load.start()

        barrier_sem = pltpu.get_barrier_semaphore()
        for nbr in (y_nbr, x_nbr):
            pl.semaphore_signal(
                barrier_sem, inc=1,
                device_id=nbr, device_id_type=pl.DeviceIdType.MESH,
            )
        pl.semaphore_wait(barrier_sem, 2)
        load.wait()

        y_rdmas = []
        for c in range(C):
            sl = pl.ds(c * ROWS, ROWS)
            ysend[sl, :] = xf32[sl, :].astype(jnp.bfloat16)
            r = pltpu.make_async_remote_copy(
                src_ref=ysend.at[sl],
                dst_ref=yrecv.at[sl],
                send_sem=ysend_sems.at[c],
                recv_sem=yrecv_sems.at[c],
                device_id=y_nbr,
                device_id_type=pl.DeviceIdType.MESH,
            )
            r.start()
            y_rdmas.append(r)

        x_rdmas = []
        for c in range(C):
            sl = pl.ds(c * ROWS, ROWS)
            out_sl = pl.ds(row0 + c * ROWS, ROWS)
            y_rdmas[c].wait_recv()
            out_ref[out_sl, :] = ysend[sl, :] + yrecv[sl, :]
            r = pltpu.make_async_remote_copy(
                src_ref=out_ref.at[out_sl],
                dst_ref=out_ref.at[out_sl],
                send_sem=xsend_sems.at[c],
                recv_sem=xrecv_sems.at[c],
                device_id=x_nbr,
                device_id_type=pl.DeviceIdType.MESH,
            )
            r.start()
            x_rdmas.append(r)

        for c in range(C):
            recv = pltpu.make_async_remote_copy(
                src_ref=out_ref.at[pl.ds(row0, ROWS)],
                dst_ref=out_ref.at[pl.ds(other_row0 + c * ROWS, ROWS)],
                send_sem=xsend_sems.at[c],
                recv_sem=xrecv_sems.at[c],
                device_id=x_nbr,
                device_id_type=pl.DeviceIdType.MESH,
            )
            recv.wait_recv()
        for c in range(C):
            y_rdmas[c].wait_send()
            x_rdmas[c].wait_send()

    return pl.pallas_call(
        body,
        out_shape=jax.ShapeDtypeStruct((M, N), jnp.bfloat16),
        in_specs=[pl.BlockSpec(memory_space=pl.ANY)],
        out_specs=pl.BlockSpec(memory_space=pltpu.VMEM),
        scratch_shapes=[
            pltpu.VMEM((HALF, N), jnp.float32),
            pltpu.VMEM((HALF, N), jnp.bfloat16),
            pltpu.VMEM((HALF, N), jnp.bfloat16),
            pltpu.SemaphoreType.DMA,
            pltpu.SemaphoreType.DMA((C,)),
            pltpu.SemaphoreType.DMA((C,)),
            pltpu.SemaphoreType.DMA((C,)),
            pltpu.SemaphoreType.DMA((C,)),
        ],
        compiler_params=pltpu.CompilerParams(collective_id=0),
    )(x)


# device time: 106888 ns/iter; 1.8611x vs baseline; 1.0611x over previous
import jax
import jax.numpy as jnp
from jax import lax
from jax.experimental import pallas as pl
from jax.experimental.pallas import tpu as pltpu


def kernel(x):
    M, N = x.shape
    HALF = M // 2

    C = 32
    ROWS = HALF // C

    def body(x_hbm, out_ref, xf32, ysend, yrecv,
             load_sems, ysend_sems, yrecv_sems, xsend_sems, xrecv_sems):
        my_x = lax.axis_index("x")
        my_y = lax.axis_index("y")
        y_nbr = (my_x, 1 - my_y)
        x_nbr = (1 - my_x, my_y)

        row0 = my_x * HALF
        other_row0 = (1 - my_x) * HALF

        loads = []
        for c in range(C):
            sl = pl.ds(c * ROWS, ROWS)
            ld = pltpu.make_async_copy(
                x_hbm.at[pl.ds(row0 + c * ROWS, ROWS)], xf32.at[sl],
                load_sems.at[c],
            )
            ld.start()
            loads.append(ld)

        barrier_sem = pltpu.get_barrier_semaphore()
        for nbr in (y_nbr, x_nbr):
            pl.semaphore_signal(
                barrier_sem, inc=1,
                device_id=nbr, device_id_type=pl.DeviceIdType.MESH,
            )
        pl.semaphore_wait(barrier_sem, 2)

        y_rdmas = []
        for c in range(C):
            sl = pl.ds(c * ROWS, ROWS)
            loads[c].wait()
            ysend[sl, :] = xf32[sl, :].astype(jnp.bfloat16)
            r = pltpu.make_async_remote_copy(
                src_ref=ysend.at[sl],
                dst_ref=yrecv.at[sl],
                send_sem=ysend_sems.at[c],
                recv_sem=yrecv_sems.at[c],
                device_id=y_nbr,
                device_id_type=pl.DeviceIdType.MESH,
            )
            r.start()
            y_rdmas.append(r)

        x_rdmas = []
        for c in range(C):
            sl = pl.ds(c * ROWS, ROWS)
            out_sl = pl.ds(row0 + c * ROWS, ROWS)
            y_rdmas[c].wait_recv()
            out_ref[out_sl, :] = ysend[sl, :] + yrecv[sl, :]
            r = pltpu.make_async_remote_copy(
                src_ref=out_ref.at[out_sl],
                dst_ref=out_ref.at[out_sl],
                send_sem=xsend_sems.at[c],
                recv_sem=xrecv_sems.at[c],
                device_id=x_nbr,
                device_id_type=pl.DeviceIdType.MESH,
            )
            r.start()
            x_rdmas.append(r)

        for c in range(C):
            recv = pltpu.make_async_remote_copy(
                src_ref=out_ref.at[pl.ds(row0, ROWS)],
                dst_ref=out_ref.at[pl.ds(other_row0 + c * ROWS, ROWS)],
                send_sem=xsend_sems.at[c],
                recv_sem=xrecv_sems.at[c],
                device_id=x_nbr,
                device_id_type=pl.DeviceIdType.MESH,
            )
            recv.wait_recv()
        for c in range(C):
            y_rdmas[c].wait_send()
            x_rdmas[c].wait_send()

    return pl.pallas_call(
        body,
        out_shape=jax.ShapeDtypeStruct((M, N), jnp.bfloat16),
        in_specs=[pl.BlockSpec(memory_space=pl.ANY)],
        out_specs=pl.BlockSpec(memory_space=pltpu.VMEM),
        scratch_shapes=[
            pltpu.VMEM((HALF, N), jnp.float32),
            pltpu.VMEM((HALF, N), jnp.bfloat16),
            pltpu.VMEM((HALF, N), jnp.bfloat16),
            pltpu.SemaphoreType.DMA((C,)),
            pltpu.SemaphoreType.DMA((C,)),
            pltpu.SemaphoreType.DMA((C,)),
            pltpu.SemaphoreType.DMA((C,)),
            pltpu.SemaphoreType.DMA((C,)),
        ],
        compiler_params=pltpu.CompilerParams(collective_id=0),
    )(x)
